# baseline (speedup 1.0000x reference)
import functools

import jax
import jax.numpy as jnp
import numpy as np
from jax.experimental import pallas as pl


_PROB = 0.25


@functools.lru_cache(maxsize=None)
def _drop_indices(b: int) -> tuple[int, ...]:
    n = 1 if b == 1 else int(b * _PROB)
    with jax.ensure_compile_time_eval():
        perm = np.asarray(jax.random.permutation(jax.random.key(1), b))
    return tuple(int(i) for i in perm[:n])


def kernel(emb0):
    b, t, c, d = emb0.shape
    drop = _drop_indices(b)
    tb = 256
    assert t % tb == 0

    def body(x_ref, o_ref):
        bi = pl.program_id(0)
        ti = pl.program_id(1)
        dropped = functools.reduce(
            jnp.logical_or, [bi == i for i in drop], jnp.bool_(False)
        )
        tt = jax.lax.broadcasted_iota(jnp.int32, x_ref.shape, 1) + ti * tb
        keep = jnp.logical_or(jnp.logical_not(dropped), tt == t - 1)
        o_ref[...] = jnp.where(keep, x_ref[...], 0.0)

    return pl.pallas_call(
        body,
        grid=(b, t // tb),
        in_specs=[pl.BlockSpec((1, tb, c, d), lambda i, j: (i, j, 0, 0))],
        out_specs=pl.BlockSpec((1, tb, c, d), lambda i, j: (i, j, 0, 0)),
        out_shape=jax.ShapeDtypeStruct(emb0.shape, emb0.dtype),
    )(emb0)
